# baseline (speedup 1.0000x reference)
import functools

import jax
import jax.numpy as jnp
from jax import lax
from jax.experimental import pallas as pl
from jax.experimental.pallas import tpu as pltpu
from jax.experimental.pallas import tpu_sc as plsc

N = 10000
E = 320000
D = 128

NC = 2
NS = 16
NW = NC * NS
EPW = E // NW
CH = 80
NCH0 = EPW // CH
NCH = 128
SEG = 32
NSEG = NCH // SEG
NBUF = 4
ROWS_PER_TILE = 624
REM_LO = NS * ROWS_PER_TILE
REM_ROWS = N - REM_LO
ZR = 8


def _sc_agg_body(h_hbm, src_hbm, dst_hbm, out_hbm,
                 src_v, dst_v, rows_v, zbuf_v, acc_sh, *sems):
    gsem = sems[:NBUF]
    ssem = sems[NBUF:]
    c = lax.axis_index("c")
    s = lax.axis_index("s")
    w = c * NS + s

    @pl.loop(0, ZR)
    def _zrow(i):
        for j in range(D // 16):
            zbuf_v[i, pl.ds(j * 16, 16)] = jnp.zeros((16,), jnp.float32)

    row_lo = s * ROWS_PER_TILE

    @pl.loop(0, ROWS_PER_TILE // ZR)
    def _zcopy(k):
        pltpu.sync_copy(zbuf_v, acc_sh.at[pl.ds(row_lo + k * ZR, ZR)])

    @pl.when(s == NS - 1)
    def _zrem():
        pltpu.sync_copy(zbuf_v, acc_sh.at[pl.ds(REM_LO, ZR)])
        pltpu.sync_copy(zbuf_v, acc_sh.at[pl.ds(REM_LO + ZR, ZR)])

    plsc.subcore_barrier()

    def _gather(k, b):
        return pltpu.async_copy(h_hbm.at[src_v.at[k]], rows_v.at[b], gsem[b])

    def _scatter(k, b):
        return pltpu.async_copy(rows_v.at[b], acc_sh.at[dst_v.at[k]],
                                ssem[b], add=True)

    @pl.loop(0, NSEG)
    def _seg(g):
        pltpu.sync_copy(src_hbm.at[w, pl.ds(g * SEG, SEG)], src_v)
        pltpu.sync_copy(dst_hbm.at[w, pl.ds(g * SEG, SEG)], dst_v)

        @pl.loop(0, SEG, step=NBUF)
        def _chunk(i):
            g0 = _gather(i + 0, 0)
            g1 = _gather(i + 1, 1)
            g0.wait()
            s0 = _scatter(i + 0, 0)
            g2 = _gather(i + 2, 2)
            g1.wait()
            s1 = _scatter(i + 1, 1)
            s0.wait()
            g3 = _gather(i + 3, 3)
            g2.wait()
            s2 = _scatter(i + 2, 2)
            s1.wait()
            g3.wait()
            s3 = _scatter(i + 3, 3)
            s2.wait()
            s3.wait()

    plsc.subcore_barrier()

    pltpu.sync_copy(acc_sh.at[pl.ds(row_lo, ROWS_PER_TILE)],
                    out_hbm.at[pl.ds(c * N + row_lo, ROWS_PER_TILE)])

    @pl.when(s == NS - 1)
    def _orem():
        pltpu.sync_copy(acc_sh.at[pl.ds(REM_LO, REM_ROWS)],
                        out_hbm.at[pl.ds(c * N + REM_LO, REM_ROWS)])


_sc_agg = functools.partial(
    pl.kernel,
    out_type=jax.ShapeDtypeStruct((2 * N, D), jnp.float32),
    mesh=plsc.VectorSubcoreMesh(core_axis_name="c", subcore_axis_name="s"),
    scratch_types=[
        pltpu.VMEM((SEG, CH), jnp.int32),
        pltpu.VMEM((SEG, CH), jnp.int32),
        pltpu.VMEM((NBUF, CH, D), jnp.float32),
        pltpu.VMEM((ZR, D), jnp.float32),
        pltpu.VMEM_SHARED((N, D), jnp.float32),
    ] + [pltpu.SemaphoreType.DMA] * (2 * NBUF),
)(_sc_agg_body)


def _tc_body(h_ref, p_ref, wa_ref, ba_ref, wb_ref, bb_ref, g_ref, be_ref,
             o_ref):
    x = h_ref[:N, :] + p_ref[:N, :] + p_ref[N:, :]
    h1 = jnp.dot(x, wa_ref[...], preferred_element_type=jnp.float32)
    h1 = jnp.maximum(h1 + ba_ref[...], 0.0)
    h2 = jnp.dot(h1, wb_ref[...], preferred_element_type=jnp.float32)
    h2 = h2 + bb_ref[...]
    mu = jnp.mean(h2, axis=0, keepdims=True)
    d = h2 - mu
    var = jnp.mean(d * d, axis=0, keepdims=True)
    o_ref[:N, :] = d * lax.rsqrt(var + 1e-5) * g_ref[...] + be_ref[...]
    o_ref[N:, :] = jnp.zeros((8, D), jnp.float32)


def _tc_stage(h, parts, Wa, ba, Wb, bb, g, be):
    return pl.pallas_call(
        _tc_body,
        out_shape=jax.ShapeDtypeStruct((N + 8, D), jnp.float32),
    )(h, parts, Wa, ba.reshape(1, D), Wb, bb.reshape(1, D),
      g.reshape(1, D), be.reshape(1, D))


def kernel(x, edge_index, batch, W0a, b0a, W0b, b0b, g0, be0,
           W1a, b1a, W1b, b1b, g1, be1,
           W2a, b2a, W2b, b2b, g2, be2):
    npad = NCH - NCH0
    pad_src = jnp.full((NW, npad, CH), N, dtype=jnp.int32)
    pad_dst = jnp.reshape(
        jnp.arange(NW * npad * CH, dtype=jnp.int32) % N, (NW, npad, CH))
    src = jnp.concatenate([edge_index[0].reshape(NW, NCH0, CH), pad_src],
                          axis=1)
    dst = jnp.concatenate([edge_index[1].reshape(NW, NCH0, CH), pad_dst],
                          axis=1)
    params = [(W0a, b0a, W0b, b0b, g0, be0),
              (W1a, b1a, W1b, b1b, g1, be1),
              (W2a, b2a, W2b, b2b, g2, be2)]
    h = jnp.concatenate([x, jnp.zeros((8, D), jnp.float32)], axis=0)
    for (Wa, ba, Wb, bb, g, b) in params:
        parts = _sc_agg(h, src, dst)
        h = _tc_stage(h, parts, Wa, ba, Wb, bb, g, b)
    return h[:N]


# speedup vs baseline: 2.6943x; 2.6943x over previous
import functools

import jax
import jax.numpy as jnp
from jax import lax
from jax.experimental import pallas as pl
from jax.experimental.pallas import tpu as pltpu
from jax.experimental.pallas import tpu_sc as plsc

N = 10000
E = 320000
D = 128

NC = 2
NS = 16
NW = NC * NS
EPW = E // NW
CH = 125
NCH = EPW // CH
SEG = 40
NSEG = NCH // SEG
ROWS_PER_TILE = 624
REM_LO = NS * ROWS_PER_TILE
REM_ROWS = N - REM_LO
ZR = 16
NZC = ROWS_PER_TILE // ZR


def _sc_agg_body(h_hbm, src_hbm, dst_hbm, out_hbm,
                 src_v, dst_v, rows_v, zbuf_v, acc_sh,
                 gsem0, gsem1, ssem0, ssem1, zsem):
    c = lax.axis_index("c")
    s = lax.axis_index("s")
    w = c * NS + s

    @pl.loop(0, ZR)
    def _zrow(i):
        for j in range(D // 16):
            zbuf_v[i, pl.ds(j * 16, 16)] = jnp.zeros((16,), jnp.float32)

    row_lo = s * ROWS_PER_TILE

    zcps = [
        pltpu.async_copy(zbuf_v, acc_sh.at[pl.ds(row_lo + k * ZR, ZR)], zsem)
        for k in range(NZC)
    ]

    @pl.when(s == NS - 1)
    def _zrem():
        cp = pltpu.async_copy(zbuf_v, acc_sh.at[pl.ds(REM_LO, REM_ROWS)],
                              zsem)
        cp.wait()

    for cp in zcps:
        cp.wait()

    idx_base = w * NCH

    plsc.subcore_barrier()

    def _gather(k, b, sem):
        return pltpu.async_copy(h_hbm.at[src_v.at[k]], rows_v.at[b], sem)

    def _scatter(k, b, sem):
        return pltpu.async_copy(rows_v.at[b], acc_sh.at[dst_v.at[k]],
                                sem, add=True)

    @pl.loop(0, NSEG)
    def _seg(g):
        pltpu.sync_copy(src_hbm.at[pl.ds(idx_base + g * SEG, SEG)], src_v)
        pltpu.sync_copy(dst_hbm.at[pl.ds(idx_base + g * SEG, SEG)], dst_v)

        @pl.loop(0, SEG, step=2)
        def _chunk(i):
            g0 = _gather(i, 0, gsem0)
            g1 = _gather(i + 1, 1, gsem1)
            g0.wait()
            s0 = _scatter(i, 0, ssem0)
            g1.wait()
            s1 = _scatter(i + 1, 1, ssem1)
            s0.wait()
            s1.wait()

    plsc.subcore_barrier()

    o0 = pltpu.async_copy(acc_sh.at[pl.ds(row_lo, ROWS_PER_TILE)],
                          out_hbm.at[pl.ds(c * N + row_lo, ROWS_PER_TILE)],
                          gsem0)

    @pl.when(s == NS - 1)
    def _orem():
        cp = pltpu.async_copy(acc_sh.at[pl.ds(REM_LO, REM_ROWS)],
                              out_hbm.at[pl.ds(c * N + REM_LO, REM_ROWS)],
                              gsem1)
        cp.wait()

    o0.wait()


_sc_agg = functools.partial(
    pl.kernel,
    out_type=jax.ShapeDtypeStruct((2 * N, D), jnp.float32),
    mesh=plsc.VectorSubcoreMesh(core_axis_name="c", subcore_axis_name="s"),
    scratch_types=[
        pltpu.VMEM((SEG, CH), jnp.int32),
        pltpu.VMEM((SEG, CH), jnp.int32),
        pltpu.VMEM((2, CH, D), jnp.float32),
        pltpu.VMEM((ZR, D), jnp.float32),
        pltpu.VMEM_SHARED((N, D), jnp.float32),
        pltpu.SemaphoreType.DMA,
        pltpu.SemaphoreType.DMA,
        pltpu.SemaphoreType.DMA,
        pltpu.SemaphoreType.DMA,
        pltpu.SemaphoreType.DMA,
    ],
)(_sc_agg_body)


def _tc_body(h_ref, p_ref, wa_ref, ba_ref, wb_ref, bb_ref, g_ref, be_ref,
             o_ref):
    x = h_ref[...] + p_ref[:N, :] + p_ref[N:, :]
    h1 = jnp.dot(x, wa_ref[...], preferred_element_type=jnp.float32)
    h1 = jnp.maximum(h1 + ba_ref[...], 0.0)
    h2 = jnp.dot(h1, wb_ref[...], preferred_element_type=jnp.float32)
    h2 = h2 + bb_ref[...]
    mu = jnp.mean(h2, axis=0, keepdims=True)
    d = h2 - mu
    var = jnp.mean(d * d, axis=0, keepdims=True)
    o_ref[...] = d * lax.rsqrt(var + 1e-5) * g_ref[...] + be_ref[...]


def _tc_stage(h, parts, Wa, ba, Wb, bb, g, be):
    return pl.pallas_call(
        _tc_body,
        out_shape=jax.ShapeDtypeStruct((N, D), jnp.float32),
    )(h, parts, Wa, ba.reshape(1, D), Wb, bb.reshape(1, D),
      g.reshape(1, D), be.reshape(1, D))


def kernel(x, edge_index, batch, W0a, b0a, W0b, b0b, g0, be0,
           W1a, b1a, W1b, b1b, g1, be1,
           W2a, b2a, W2b, b2b, g2, be2):
    src = edge_index[0].reshape(E // CH, CH)
    dst = edge_index[1].reshape(E // CH, CH)
    params = [(W0a, b0a, W0b, b0b, g0, be0),
              (W1a, b1a, W1b, b1b, g1, be1),
              (W2a, b2a, W2b, b2b, g2, be2)]
    h = x
    for (Wa, ba, Wb, bb, g, b) in params:
        parts = _sc_agg(h, src, dst)
        h = _tc_stage(h, parts, Wa, ba, Wb, bb, g, b)
    return h


# speedup vs baseline: 2.7203x; 1.0096x over previous
import functools

import jax
import jax.numpy as jnp
from jax import lax
from jax.experimental import pallas as pl
from jax.experimental.pallas import tpu as pltpu
from jax.experimental.pallas import tpu_sc as plsc

N = 10000
E = 320000
D = 128

NC = 2
NS = 16
NW = NC * NS
EPW = E // NW
CH = 125
NCH = EPW // CH
SEG = 40
NSEG = NCH // SEG
ROWS_PER_TILE = 624
REM_LO = NS * ROWS_PER_TILE
REM_ROWS = N - REM_LO
ZR = 16
NZC = ROWS_PER_TILE // ZR


def _sc_agg_body(h_hbm, src_hbm, dst_hbm, out_hbm,
                 src_v, dst_v, rows_v, zbuf_v, acc_sh,
                 gsem0, gsem1, ssem0, ssem1, zsem):
    c = lax.axis_index("c")
    s = lax.axis_index("s")
    w = c * NS + s

    @pl.loop(0, ZR)
    def _zrow(i):
        for j in range(D // 16):
            zbuf_v[i, pl.ds(j * 16, 16)] = jnp.zeros((16,), jnp.float32)

    row_lo = s * ROWS_PER_TILE

    zcps = [
        pltpu.async_copy(zbuf_v, acc_sh.at[pl.ds(row_lo + k * ZR, ZR)], zsem)
        for k in range(NZC)
    ]

    idx_base = w * NCH
    i0 = pltpu.async_copy(src_hbm.at[pl.ds(idx_base, SEG)], src_v, gsem0)
    i1 = pltpu.async_copy(dst_hbm.at[pl.ds(idx_base, SEG)], dst_v, gsem1)

    @pl.when(s == NS - 1)
    def _zrem():
        cp = pltpu.async_copy(zbuf_v, acc_sh.at[pl.ds(REM_LO, REM_ROWS)],
                              zsem)
        cp.wait()

    for cp in zcps:
        cp.wait()
    i0.wait()
    i1.wait()

    plsc.subcore_barrier()

    def _gather(k, b, sem):
        return pltpu.async_copy(h_hbm.at[src_v.at[k]], rows_v.at[b], sem)

    def _scatter(k, b, sem):
        return pltpu.async_copy(rows_v.at[b], acc_sh.at[dst_v.at[k]],
                                sem, add=True)

    def _chunk_loop():
        @pl.loop(0, SEG, step=2)
        def _chunk(i):
            g0 = _gather(i, 0, gsem0)
            g1 = _gather(i + 1, 1, gsem1)
            g0.wait()
            s0 = _scatter(i, 0, ssem0)
            g1.wait()
            s1 = _scatter(i + 1, 1, ssem1)
            s0.wait()
            s1.wait()

    _chunk_loop()
    pltpu.sync_copy(src_hbm.at[pl.ds(idx_base + SEG, SEG)], src_v)
    pltpu.sync_copy(dst_hbm.at[pl.ds(idx_base + SEG, SEG)], dst_v)
    _chunk_loop()

    plsc.subcore_barrier()

    o0 = pltpu.async_copy(acc_sh.at[pl.ds(row_lo, ROWS_PER_TILE)],
                          out_hbm.at[pl.ds(c * N + row_lo, ROWS_PER_TILE)],
                          gsem0)

    @pl.when(s == NS - 1)
    def _orem():
        cp = pltpu.async_copy(acc_sh.at[pl.ds(REM_LO, REM_ROWS)],
                              out_hbm.at[pl.ds(c * N + REM_LO, REM_ROWS)],
                              gsem1)
        cp.wait()

    o0.wait()


_sc_agg = functools.partial(
    pl.kernel,
    out_type=jax.ShapeDtypeStruct((2 * N, D), jnp.float32),
    mesh=plsc.VectorSubcoreMesh(core_axis_name="c", subcore_axis_name="s"),
    scratch_types=[
        pltpu.VMEM((SEG, CH), jnp.int32),
        pltpu.VMEM((SEG, CH), jnp.int32),
        pltpu.VMEM((2, CH, D), jnp.float32),
        pltpu.VMEM((ZR, D), jnp.float32),
        pltpu.VMEM_SHARED((N, D), jnp.float32),
        pltpu.SemaphoreType.DMA,
        pltpu.SemaphoreType.DMA,
        pltpu.SemaphoreType.DMA,
        pltpu.SemaphoreType.DMA,
        pltpu.SemaphoreType.DMA,
    ],
)(_sc_agg_body)


def _tc_body(h_ref, p_ref, wa_ref, ba_ref, wb_ref, bb_ref, g_ref, be_ref,
             o_ref):
    x = h_ref[...] + p_ref[:N, :] + p_ref[N:, :]
    h1 = jnp.dot(x, wa_ref[...], preferred_element_type=jnp.float32)
    h1 = jnp.maximum(h1 + ba_ref[...], 0.0)
    h2 = jnp.dot(h1, wb_ref[...], preferred_element_type=jnp.float32)
    h2 = h2 + bb_ref[...]
    mu = jnp.mean(h2, axis=0, keepdims=True)
    d = h2 - mu
    var = jnp.mean(d * d, axis=0, keepdims=True)
    o_ref[...] = d * lax.rsqrt(var + 1e-5) * g_ref[...] + be_ref[...]


def _tc_stage(h, parts, Wa, ba, Wb, bb, g, be):
    return pl.pallas_call(
        _tc_body,
        out_shape=jax.ShapeDtypeStruct((N, D), jnp.float32),
    )(h, parts, Wa, ba.reshape(1, D), Wb, bb.reshape(1, D),
      g.reshape(1, D), be.reshape(1, D))


def kernel(x, edge_index, batch, W0a, b0a, W0b, b0b, g0, be0,
           W1a, b1a, W1b, b1b, g1, be1,
           W2a, b2a, W2b, b2b, g2, be2):
    src = edge_index[0].reshape(E // CH, CH)
    dst = edge_index[1].reshape(E // CH, CH)
    params = [(W0a, b0a, W0b, b0b, g0, be0),
              (W1a, b1a, W1b, b1b, g1, be1),
              (W2a, b2a, W2b, b2b, g2, be2)]
    h = x
    for (Wa, ba, Wb, bb, g, b) in params:
        parts = _sc_agg(h, src, dst)
        h = _tc_stage(h, parts, Wa, ba, Wb, bb, g, b)
    return h


# speedup vs baseline: 2.7427x; 1.0082x over previous
import functools

import jax
import jax.numpy as jnp
from jax import lax
from jax.experimental import pallas as pl
from jax.experimental.pallas import tpu as pltpu
from jax.experimental.pallas import tpu_sc as plsc

N = 10000
E = 320000
D = 128

NC = 2
NS = 16
NW = NC * NS
EPW = E // NW
CH = 125
NCH = EPW // CH
SEG = 40
NSEG = NCH // SEG
ROWS_PER_TILE = 624
REM_LO = NS * ROWS_PER_TILE
REM_ROWS = N - REM_LO
ZR = 16
NZC = ROWS_PER_TILE // ZR


def _sc_agg_body(h_hbm, src_hbm, dst_hbm, out_hbm,
                 src_v, dst_v, rows_v, zbuf_v, acc_sh,
                 gsem0, gsem1, ssem0, ssem1, zsem):
    c = lax.axis_index("c")
    s = lax.axis_index("s")
    w = c * NS + s

    @pl.loop(0, ZR)
    def _zrow(i):
        for j in range(D // 16):
            zbuf_v[i, pl.ds(j * 16, 16)] = jnp.zeros((16,), jnp.float32)

    row_lo = s * ROWS_PER_TILE

    zcps = [
        pltpu.async_copy(zbuf_v, acc_sh.at[pl.ds(row_lo + k * ZR, ZR)], zsem)
        for k in range(NZC)
    ]

    idx_base = w * NCH
    i0 = pltpu.async_copy(src_hbm.at[pl.ds(idx_base, SEG)], src_v, gsem0)
    i1 = pltpu.async_copy(dst_hbm.at[pl.ds(idx_base, SEG)], dst_v, gsem1)

    @pl.when(s == NS - 1)
    def _zrem():
        cp = pltpu.async_copy(zbuf_v, acc_sh.at[pl.ds(REM_LO, REM_ROWS)],
                              zsem)
        cp.wait()

    for cp in zcps:
        cp.wait()
    i0.wait()
    i1.wait()

    plsc.subcore_barrier()

    def _gather(k, b, sem):
        return pltpu.async_copy(h_hbm.at[src_v.at[k]], rows_v.at[b], sem)

    def _scatter(k, b, sem):
        return pltpu.async_copy(rows_v.at[b], acc_sh.at[dst_v.at[k]],
                                sem, add=True)

    def _chunk_loop():
        @pl.loop(0, SEG, step=4)
        def _chunk(i):
            g0 = _gather(i, 0, gsem0)
            g1 = _gather(i + 1, 1, gsem1)
            g0.wait()
            s0 = _scatter(i, 0, ssem0)
            g1.wait()
            s1 = _scatter(i + 1, 1, ssem1)
            s0.wait()
            g2 = _gather(i + 2, 0, gsem0)
            s1.wait()
            g3 = _gather(i + 3, 1, gsem1)
            g2.wait()
            s2 = _scatter(i + 2, 0, ssem0)
            g3.wait()
            s3 = _scatter(i + 3, 1, ssem1)
            s2.wait()
            s3.wait()

    _chunk_loop()
    pltpu.sync_copy(src_hbm.at[pl.ds(idx_base + SEG, SEG)], src_v)
    pltpu.sync_copy(dst_hbm.at[pl.ds(idx_base + SEG, SEG)], dst_v)
    _chunk_loop()

    plsc.subcore_barrier()

    o0 = pltpu.async_copy(acc_sh.at[pl.ds(row_lo, ROWS_PER_TILE)],
                          out_hbm.at[pl.ds(c * N + row_lo, ROWS_PER_TILE)],
                          gsem0)

    @pl.when(s == NS - 1)
    def _orem():
        cp = pltpu.async_copy(acc_sh.at[pl.ds(REM_LO, REM_ROWS)],
                              out_hbm.at[pl.ds(c * N + REM_LO, REM_ROWS)],
                              gsem1)
        cp.wait()

    o0.wait()


_sc_agg = functools.partial(
    pl.kernel,
    out_type=jax.ShapeDtypeStruct((2 * N, D), jnp.float32),
    mesh=plsc.VectorSubcoreMesh(core_axis_name="c", subcore_axis_name="s"),
    scratch_types=[
        pltpu.VMEM((SEG, CH), jnp.int32),
        pltpu.VMEM((SEG, CH), jnp.int32),
        pltpu.VMEM((2, CH, D), jnp.float32),
        pltpu.VMEM((ZR, D), jnp.float32),
        pltpu.VMEM_SHARED((N, D), jnp.float32),
        pltpu.SemaphoreType.DMA,
        pltpu.SemaphoreType.DMA,
        pltpu.SemaphoreType.DMA,
        pltpu.SemaphoreType.DMA,
        pltpu.SemaphoreType.DMA,
    ],
)(_sc_agg_body)


def _tc_body(h_ref, p_ref, wa_ref, ba_ref, wb_ref, bb_ref, g_ref, be_ref,
             o_ref):
    x = h_ref[...] + p_ref[:N, :] + p_ref[N:, :]
    h1 = jnp.dot(x, wa_ref[...], preferred_element_type=jnp.float32)
    h1 = jnp.maximum(h1 + ba_ref[...], 0.0)
    h2 = jnp.dot(h1, wb_ref[...], preferred_element_type=jnp.float32)
    h2 = h2 + bb_ref[...]
    mu = jnp.mean(h2, axis=0, keepdims=True)
    d = h2 - mu
    var = jnp.mean(d * d, axis=0, keepdims=True)
    o_ref[...] = d * lax.rsqrt(var + 1e-5) * g_ref[...] + be_ref[...]


def _tc_stage(h, parts, Wa, ba, Wb, bb, g, be):
    return pl.pallas_call(
        _tc_body,
        out_shape=jax.ShapeDtypeStruct((N, D), jnp.float32),
    )(h, parts, Wa, ba.reshape(1, D), Wb, bb.reshape(1, D),
      g.reshape(1, D), be.reshape(1, D))


def kernel(x, edge_index, batch, W0a, b0a, W0b, b0b, g0, be0,
           W1a, b1a, W1b, b1b, g1, be1,
           W2a, b2a, W2b, b2b, g2, be2):
    src = edge_index[0].reshape(E // CH, CH)
    dst = edge_index[1].reshape(E // CH, CH)
    params = [(W0a, b0a, W0b, b0b, g0, be0),
              (W1a, b1a, W1b, b1b, g1, be1),
              (W2a, b2a, W2b, b2b, g2, be2)]
    h = x
    for (Wa, ba, Wb, bb, g, b) in params:
        parts = _sc_agg(h, src, dst)
        h = _tc_stage(h, parts, Wa, ba, Wb, bb, g, b)
    return h


# speedup vs baseline: 2.7605x; 1.0065x over previous
import functools

import jax
import jax.numpy as jnp
from jax import lax
from jax.experimental import pallas as pl
from jax.experimental.pallas import tpu as pltpu
from jax.experimental.pallas import tpu_sc as plsc

N = 10000
E = 320000
D = 128

NC = 2
NS = 16
NW = NC * NS
EPW = E // NW
CH = 125
NCH = EPW // CH
SEG = 40
NSEG = NCH // SEG
ROWS_PER_TILE = 624
REM_LO = NS * ROWS_PER_TILE
REM_ROWS = N - REM_LO
ZR = 16
NZC = ROWS_PER_TILE // ZR


def _sc_agg_body(h_hbm, src_hbm, dst_hbm, out_hbm,
                 src_v, dst_v, rows_v, zbuf_v, acc_sh,
                 gsem0, gsem1, ssem0, ssem1, zsem):
    c = lax.axis_index("c")
    s = lax.axis_index("s")
    w = c * NS + s

    @pl.loop(0, ZR)
    def _zrow(i):
        for j in range(D // 16):
            zbuf_v[i, pl.ds(j * 16, 16)] = jnp.zeros((16,), jnp.float32)

    row_lo = s * ROWS_PER_TILE

    zcps = [
        pltpu.async_copy(zbuf_v, acc_sh.at[pl.ds(row_lo + k * ZR, ZR)], zsem)
        for k in range(NZC)
    ]

    idx_base = w * NCH
    i0 = pltpu.async_copy(src_hbm.at[pl.ds(idx_base, SEG)], src_v, gsem0)
    i1 = pltpu.async_copy(dst_hbm.at[pl.ds(idx_base, SEG)], dst_v, gsem1)

    @pl.when(s == NS - 1)
    def _zrem():
        cp = pltpu.async_copy(zbuf_v, acc_sh.at[pl.ds(REM_LO, REM_ROWS)],
                              zsem)
        cp.wait()

    for cp in zcps:
        cp.wait()
    i0.wait()
    i1.wait()

    plsc.subcore_barrier()

    def _gather(k, b, sem):
        return pltpu.async_copy(h_hbm.at[src_v.at[k]], rows_v.at[b], sem)

    def _scatter(k, b, sem):
        return pltpu.async_copy(rows_v.at[b], acc_sh.at[dst_v.at[k]],
                                sem, add=True)

    NPAIR = 4

    def _chunk_loop():
        @pl.loop(0, SEG, step=2 * NPAIR)
        def _chunk(i):
            g0 = _gather(i, 0, gsem0)
            g1 = _gather(i + 1, 1, gsem1)
            g0.wait()
            s0 = _scatter(i, 0, ssem0)
            g1.wait()
            s1 = _scatter(i + 1, 1, ssem1)
            for p in range(1, NPAIR):
                k = i + 2 * p
                s0.wait()
                g0 = _gather(k, 0, gsem0)
                s1.wait()
                g1 = _gather(k + 1, 1, gsem1)
                g0.wait()
                s0 = _scatter(k, 0, ssem0)
                g1.wait()
                s1 = _scatter(k + 1, 1, ssem1)
            s0.wait()
            s1.wait()

    _chunk_loop()
    pltpu.sync_copy(src_hbm.at[pl.ds(idx_base + SEG, SEG)], src_v)
    pltpu.sync_copy(dst_hbm.at[pl.ds(idx_base + SEG, SEG)], dst_v)
    _chunk_loop()

    plsc.subcore_barrier()

    o0 = pltpu.async_copy(acc_sh.at[pl.ds(row_lo, ROWS_PER_TILE)],
                          out_hbm.at[pl.ds(c * N + row_lo, ROWS_PER_TILE)],
                          gsem0)

    @pl.when(s == NS - 1)
    def _orem():
        cp = pltpu.async_copy(acc_sh.at[pl.ds(REM_LO, REM_ROWS)],
                              out_hbm.at[pl.ds(c * N + REM_LO, REM_ROWS)],
                              gsem1)
        cp.wait()

    o0.wait()


_sc_agg = functools.partial(
    pl.kernel,
    out_type=jax.ShapeDtypeStruct((2 * N, D), jnp.float32),
    mesh=plsc.VectorSubcoreMesh(core_axis_name="c", subcore_axis_name="s"),
    scratch_types=[
        pltpu.VMEM((SEG, CH), jnp.int32),
        pltpu.VMEM((SEG, CH), jnp.int32),
        pltpu.VMEM((2, CH, D), jnp.float32),
        pltpu.VMEM((ZR, D), jnp.float32),
        pltpu.VMEM_SHARED((N, D), jnp.float32),
        pltpu.SemaphoreType.DMA,
        pltpu.SemaphoreType.DMA,
        pltpu.SemaphoreType.DMA,
        pltpu.SemaphoreType.DMA,
        pltpu.SemaphoreType.DMA,
    ],
)(_sc_agg_body)


def _tc_body(h_ref, p_ref, wa_ref, ba_ref, wb_ref, bb_ref, g_ref, be_ref,
             o_ref):
    x = h_ref[...] + p_ref[:N, :] + p_ref[N:, :]
    h1 = jnp.dot(x, wa_ref[...], preferred_element_type=jnp.float32)
    h1 = jnp.maximum(h1 + ba_ref[...], 0.0)
    h2 = jnp.dot(h1, wb_ref[...], preferred_element_type=jnp.float32)
    h2 = h2 + bb_ref[...]
    mu = jnp.mean(h2, axis=0, keepdims=True)
    d = h2 - mu
    var = jnp.mean(d * d, axis=0, keepdims=True)
    o_ref[...] = d * lax.rsqrt(var + 1e-5) * g_ref[...] + be_ref[...]


def _tc_stage(h, parts, Wa, ba, Wb, bb, g, be):
    return pl.pallas_call(
        _tc_body,
        out_shape=jax.ShapeDtypeStruct((N, D), jnp.float32),
    )(h, parts, Wa, ba.reshape(1, D), Wb, bb.reshape(1, D),
      g.reshape(1, D), be.reshape(1, D))


def kernel(x, edge_index, batch, W0a, b0a, W0b, b0b, g0, be0,
           W1a, b1a, W1b, b1b, g1, be1,
           W2a, b2a, W2b, b2b, g2, be2):
    src = edge_index[0].reshape(E // CH, CH)
    dst = edge_index[1].reshape(E // CH, CH)
    params = [(W0a, b0a, W0b, b0b, g0, be0),
              (W1a, b1a, W1b, b1b, g1, be1),
              (W2a, b2a, W2b, b2b, g2, be2)]
    h = x
    for (Wa, ba, Wb, bb, g, b) in params:
        parts = _sc_agg(h, src, dst)
        h = _tc_stage(h, parts, Wa, ba, Wb, bb, g, b)
    return h


# speedup vs baseline: 2.7676x; 1.0026x over previous
import functools

import jax
import jax.numpy as jnp
from jax import lax
from jax.experimental import pallas as pl
from jax.experimental.pallas import tpu as pltpu
from jax.experimental.pallas import tpu_sc as plsc

N = 10000
E = 320000
D = 128

NC = 2
NS = 16
NW = NC * NS
EPW = E // NW
CH = 125
NCH = EPW // CH
SEG = 40
NSEG = NCH // SEG
ROWS_PER_TILE = 624
REM_LO = NS * ROWS_PER_TILE
REM_ROWS = N - REM_LO
ZR = 16
NZC = ROWS_PER_TILE // ZR


def _sc_agg_body(h_hbm, src_hbm, dst_hbm, out_hbm,
                 src_v, dst_v, rows_v, zbuf_v, acc_sh,
                 gsem0, gsem1, ssem0, ssem1, zsem):
    c = lax.axis_index("c")
    s = lax.axis_index("s")
    w = c * NS + s

    @pl.loop(0, ZR)
    def _zrow(i):
        for j in range(D // 16):
            zbuf_v[i, pl.ds(j * 16, 16)] = jnp.zeros((16,), jnp.float32)

    row_lo = s * ROWS_PER_TILE

    zcps = [
        pltpu.async_copy(zbuf_v, acc_sh.at[pl.ds(row_lo + k * ZR, ZR)], zsem)
        for k in range(NZC)
    ]

    idx_base = w * NCH
    i0 = pltpu.async_copy(src_hbm.at[pl.ds(idx_base, SEG)], src_v, gsem0)
    i1 = pltpu.async_copy(dst_hbm.at[pl.ds(idx_base, SEG)], dst_v, gsem1)

    @pl.when(s == NS - 1)
    def _zrem():
        cp = pltpu.async_copy(zbuf_v, acc_sh.at[pl.ds(REM_LO, REM_ROWS)],
                              zsem)
        cp.wait()

    for cp in zcps:
        cp.wait()
    i0.wait()
    i1.wait()

    plsc.subcore_barrier()

    def _gather(k, b, sem):
        return pltpu.async_copy(h_hbm.at[src_v.at[k]], rows_v.at[b], sem)

    def _scatter(k, b, sem):
        return pltpu.async_copy(rows_v.at[b], acc_sh.at[dst_v.at[k]],
                                sem, add=True)

    NPAIR = 20

    def _chunk_loop():
        @pl.loop(0, SEG, step=2 * NPAIR)
        def _chunk(i):
            g0 = _gather(i, 0, gsem0)
            g1 = _gather(i + 1, 1, gsem1)
            g0.wait()
            s0 = _scatter(i, 0, ssem0)
            g1.wait()
            s1 = _scatter(i + 1, 1, ssem1)
            for p in range(1, NPAIR):
                k = i + 2 * p
                s0.wait()
                g0 = _gather(k, 0, gsem0)
                s1.wait()
                g1 = _gather(k + 1, 1, gsem1)
                g0.wait()
                s0 = _scatter(k, 0, ssem0)
                g1.wait()
                s1 = _scatter(k + 1, 1, ssem1)
            s0.wait()
            s1.wait()

    _chunk_loop()
    pltpu.sync_copy(src_hbm.at[pl.ds(idx_base + SEG, SEG)], src_v)
    pltpu.sync_copy(dst_hbm.at[pl.ds(idx_base + SEG, SEG)], dst_v)
    _chunk_loop()

    plsc.subcore_barrier()

    o0 = pltpu.async_copy(acc_sh.at[pl.ds(row_lo, ROWS_PER_TILE)],
                          out_hbm.at[pl.ds(c * N + row_lo, ROWS_PER_TILE)],
                          gsem0)

    @pl.when(s == NS - 1)
    def _orem():
        cp = pltpu.async_copy(acc_sh.at[pl.ds(REM_LO, REM_ROWS)],
                              out_hbm.at[pl.ds(c * N + REM_LO, REM_ROWS)],
                              gsem1)
        cp.wait()

    o0.wait()


_sc_agg = functools.partial(
    pl.kernel,
    out_type=jax.ShapeDtypeStruct((2 * N, D), jnp.float32),
    mesh=plsc.VectorSubcoreMesh(core_axis_name="c", subcore_axis_name="s"),
    scratch_types=[
        pltpu.VMEM((SEG, CH), jnp.int32),
        pltpu.VMEM((SEG, CH), jnp.int32),
        pltpu.VMEM((2, CH, D), jnp.float32),
        pltpu.VMEM((ZR, D), jnp.float32),
        pltpu.VMEM_SHARED((N, D), jnp.float32),
        pltpu.SemaphoreType.DMA,
        pltpu.SemaphoreType.DMA,
        pltpu.SemaphoreType.DMA,
        pltpu.SemaphoreType.DMA,
        pltpu.SemaphoreType.DMA,
    ],
)(_sc_agg_body)


def _tc_body(h_ref, p_ref, wa_ref, ba_ref, wb_ref, bb_ref, g_ref, be_ref,
             o_ref):
    x = h_ref[...] + p_ref[:N, :] + p_ref[N:, :]
    h1 = jnp.dot(x, wa_ref[...], preferred_element_type=jnp.float32)
    h1 = jnp.maximum(h1 + ba_ref[...], 0.0)
    h2 = jnp.dot(h1, wb_ref[...], preferred_element_type=jnp.float32)
    h2 = h2 + bb_ref[...]
    mu = jnp.mean(h2, axis=0, keepdims=True)
    d = h2 - mu
    var = jnp.mean(d * d, axis=0, keepdims=True)
    o_ref[...] = d * lax.rsqrt(var + 1e-5) * g_ref[...] + be_ref[...]


def _tc_stage(h, parts, Wa, ba, Wb, bb, g, be):
    return pl.pallas_call(
        _tc_body,
        out_shape=jax.ShapeDtypeStruct((N, D), jnp.float32),
    )(h, parts, Wa, ba.reshape(1, D), Wb, bb.reshape(1, D),
      g.reshape(1, D), be.reshape(1, D))


def kernel(x, edge_index, batch, W0a, b0a, W0b, b0b, g0, be0,
           W1a, b1a, W1b, b1b, g1, be1,
           W2a, b2a, W2b, b2b, g2, be2):
    src = edge_index[0].reshape(E // CH, CH)
    dst = edge_index[1].reshape(E // CH, CH)
    params = [(W0a, b0a, W0b, b0b, g0, be0),
              (W1a, b1a, W1b, b1b, g1, be1),
              (W2a, b2a, W2b, b2b, g2, be2)]
    h = x
    for (Wa, ba, Wb, bb, g, b) in params:
        parts = _sc_agg(h, src, dst)
        h = _tc_stage(h, parts, Wa, ba, Wb, bb, g, b)
    return h
